# baseline (speedup 1.0000x reference)
import functools

import jax
import jax.numpy as jnp
from jax import lax
from jax.experimental import pallas as pl
from jax.experimental.pallas import tpu as pltpu
from jax.experimental.pallas import tpu_sc as plsc

B = 16
N = 16384
H = N // 2
NPOINT = 512
L = 16
UNROLL = 8
GROUP = L * UNROLL
STEPS = H // GROUP
BIG_I32 = 2**31 - 1
NEG_HUGE = -3.4e38


def _fps_body(x_hbm, y_hbm, z_hbm, idx_hbm, x_v, y_v, z_v, t_v, i_v,
              msg_v, rcv_v, shr, cnt):
    c = lax.axis_index("c")
    s = lax.axis_index("s")
    b = c * 8 + s // 2
    half = s % 2
    partner = s ^ 1
    off = half * H

    cnt[0] = jnp.int32(0)

    pltpu.sync_copy(x_hbm.at[b, pl.ds(off, H)], x_v.at[pl.ds(0, H)])
    pltpu.sync_copy(y_hbm.at[b, pl.ds(off, H)], y_v.at[pl.ds(0, H)])
    pltpu.sync_copy(z_hbm.at[b, pl.ds(off, H)], z_v.at[pl.ds(0, H)])

    lanes = lax.iota(jnp.int32, L)
    neg = jnp.full((L,), -1e10, jnp.float32)
    pos = jnp.full((L,), 1e10, jnp.float32)
    zero = jnp.zeros((L,), jnp.float32)

    for k in range(UNROLL):
        sl = pl.ds(H + k * L, L)
        x_v[sl] = zero
        y_v[sl] = zero
        z_v[sl] = zero
        t_v[sl] = zero

    def init_body(j, carry):
        sl = pl.ds(j * L, L)
        xs = x_v[sl]
        ys = y_v[sl]
        zs = z_v[sl]
        mag = xs * xs + ys * ys + zs * zs
        t_v[sl] = jnp.where(mag <= 0.001, neg, pos)
        return carry

    lax.fori_loop(0, H // L, init_body, 0)

    pltpu.sync_copy(x_hbm.at[b, pl.ds(0, L)], msg_v)
    cx0 = msg_v[...][0]
    pltpu.sync_copy(y_hbm.at[b, pl.ds(0, L)], msg_v)
    cy0 = msg_v[...][0]
    pltpu.sync_copy(z_hbm.at[b, pl.ds(0, L)], msg_v)
    cz0 = msg_v[...][0]

    def ld_group(g):
        base = g * GROUP
        return tuple(
            r[pl.ds(base + k * L, L)]
            for r in (x_v, y_v, z_v)
            for k in range(UNROLL))

    def outer(gi, t, carry):
        cxs, cys, czs, farg, vec = carry
        vec = jnp.where(lanes == t, jnp.full((L,), farg, jnp.int32), vec)

        def inner(j, icarry):
            m, mi, wx, wy, wz, cur = icarry
            nxt = ld_group(j + 1)
            base = j * GROUP
            for k in range(UNROLL):
                sl = pl.ds(base + k * L, L)
                xs = cur[k]
                ys = cur[UNROLL + k]
                zs = cur[2 * UNROLL + k]
                dx = xs - cxs
                dy = ys - cys
                dz = zs - czs
                d = dx * dx + dy * dy + dz * dz
                t2 = jnp.minimum(t_v[sl], d)
                t_v[sl] = t2
                gt = t2 > m
                m = jnp.where(gt, t2, m)
                mi = jnp.where(gt, base + k * L + lanes, mi)
                wx = jnp.where(gt, xs, wx)
                wy = jnp.where(gt, ys, wy)
                wz = jnp.where(gt, zs, wz)
            return m, mi, wx, wy, wz, nxt

        m0 = jnp.full((L,), NEG_HUGE, jnp.float32)
        mi0 = jnp.zeros((L,), jnp.int32)
        m, mi, wx, wy, wz, _ = lax.fori_loop(
            0, STEPS, inner, (m0, mi0, zero, zero, zero, ld_group(0)))

        mx = jnp.max(m)
        lmin = jnp.min(jnp.where(m == mx, mi, BIG_I32))
        gidx = lmin + off
        win = mi == lmin
        nh = jnp.full((L,), NEG_HUGE, jnp.float32)
        px = jnp.max(jnp.where(win, wx, nh))
        py = jnp.max(jnp.where(win, wy, nh))
        pz = jnp.max(jnp.where(win, wz, nh))

        par = gi % 2
        gf = plsc.bitcast(jnp.full((L,), gidx, jnp.int32), jnp.float32)
        msg = jnp.where(lanes == 0, jnp.full((L,), mx, jnp.float32),
              jnp.where(lanes == 1, gf,
              jnp.where(lanes == 2, jnp.full((L,), px, jnp.float32),
              jnp.where(lanes == 3, jnp.full((L,), py, jnp.float32),
                        jnp.full((L,), pz, jnp.float32)))))
        msg_v[...] = msg
        pltpu.sync_copy(msg_v, shr.at[pl.ds(par * (16 * L) + s * L, L)])
        plsc.fetch_and_add(cnt.at[0], jnp.int32(1), subcore_id=partner)

        lax.while_loop(
            lambda v: v <= gi,
            lambda v: plsc.fetch_and_add(cnt.at[0], jnp.int32(0),
                                         subcore_id=s),
            jnp.int32(0))

        pltpu.sync_copy(shr.at[pl.ds(par * (16 * L) + partner * L, L)],
                        rcv_v)
        r = rcv_v[...]
        ri = plsc.bitcast(r, jnp.int32)

        m2 = r[0]
        g2 = ri[1]
        take2 = (m2 > mx) | ((m2 == mx) & (g2 < gidx))
        farg2 = jnp.where(take2, g2, gidx)
        cx2 = jnp.full((L,), jnp.where(take2, r[2], px), jnp.float32)
        cy2 = jnp.full((L,), jnp.where(take2, r[3], py), jnp.float32)
        cz2 = jnp.full((L,), jnp.where(take2, r[4], pz), jnp.float32)
        return cx2, cy2, cz2, farg2, vec

    def blk_body(blk, carry):
        cxs, cys, czs, farg = carry

        def sub(t, sc):
            return outer(blk * L + t, t, sc)

        cxs, cys, czs, farg, vec = lax.fori_loop(
            0, L, sub,
            (cxs, cys, czs, farg, jnp.zeros((L,), jnp.int32)))

        @pl.when(half == 0)
        def _():
            i_v[pl.ds(blk * L, L)] = vec

        return cxs, cys, czs, farg

    carry0 = (jnp.full((L,), cx0, jnp.float32),
              jnp.full((L,), cy0, jnp.float32),
              jnp.full((L,), cz0, jnp.float32),
              jnp.int32(0))
    lax.fori_loop(0, NPOINT // L, blk_body, carry0)

    @pl.when(half == 0)
    def _():
        pltpu.sync_copy(i_v, idx_hbm.at[b])


_fps = functools.partial(
    pl.kernel,
    out_type=jax.ShapeDtypeStruct((B, NPOINT), jnp.int32),
    mesh=plsc.VectorSubcoreMesh(core_axis_name="c", subcore_axis_name="s"),
    compiler_params=pltpu.CompilerParams(needs_layout_passes=False),
    scratch_types=[
        pltpu.VMEM((H + GROUP,), jnp.float32),
        pltpu.VMEM((H + GROUP,), jnp.float32),
        pltpu.VMEM((H + GROUP,), jnp.float32),
        pltpu.VMEM((H + GROUP,), jnp.float32),
        pltpu.VMEM((NPOINT,), jnp.int32),
        pltpu.VMEM((L,), jnp.float32),
        pltpu.VMEM((L,), jnp.float32),
        pltpu.VMEM_SHARED((2 * 16 * L,), jnp.float32),
        pltpu.SMEM((1,), jnp.int32),
    ],
)(_fps_body)


def kernel(xyz, npoint):
    del npoint
    x = xyz[:, :, 0]
    y = xyz[:, :, 1]
    z = xyz[:, :, 2]
    return _fps(x, y, z)


# speedup vs baseline: 1.4075x; 1.4075x over previous
import functools

import jax
import jax.numpy as jnp
from jax import lax
from jax.experimental import pallas as pl
from jax.experimental.pallas import tpu as pltpu
from jax.experimental.pallas import tpu_sc as plsc

B = 16
N = 16384
H = N // 2
NPOINT = 512
L = 16
UNROLL = 4
GROUP = L * UNROLL
STEPS = H // GROUP
BIG_I32 = 2**31 - 1
NEG_HUGE = -3.4e38


def _fps_body(x_hbm, y_hbm, z_hbm, idx_hbm, x_v, y_v, z_v, t_v, i_v,
              msg_v, rcv_v, shr, cnt):
    c = lax.axis_index("c")
    s = lax.axis_index("s")
    b = c * 8 + s // 2
    half = s % 2
    partner = s ^ 1
    off = half * H

    cnt[0] = jnp.int32(0)

    pltpu.sync_copy(x_hbm.at[b, pl.ds(off, H)], x_v.at[pl.ds(0, H)])
    pltpu.sync_copy(y_hbm.at[b, pl.ds(off, H)], y_v.at[pl.ds(0, H)])
    pltpu.sync_copy(z_hbm.at[b, pl.ds(off, H)], z_v.at[pl.ds(0, H)])

    lanes = lax.iota(jnp.int32, L)
    neg = jnp.full((L,), -1e10, jnp.float32)
    pos = jnp.full((L,), 1e10, jnp.float32)
    zero = jnp.zeros((L,), jnp.float32)

    for k in range(UNROLL):
        sl = pl.ds(H + k * L, L)
        x_v[sl] = zero
        y_v[sl] = zero
        z_v[sl] = zero
        t_v[sl] = zero

    def init_body(j, carry):
        sl = pl.ds(j * L, L)
        xs = x_v[sl]
        ys = y_v[sl]
        zs = z_v[sl]
        mag = xs * xs + ys * ys + zs * zs
        t_v[sl] = jnp.where(mag <= 0.001, neg, pos)
        return carry

    lax.fori_loop(0, H // L, init_body, 0)

    pltpu.sync_copy(x_hbm.at[b, pl.ds(0, L)], msg_v)
    cx0 = msg_v[...][0]
    pltpu.sync_copy(y_hbm.at[b, pl.ds(0, L)], msg_v)
    cy0 = msg_v[...][0]
    pltpu.sync_copy(z_hbm.at[b, pl.ds(0, L)], msg_v)
    cz0 = msg_v[...][0]

    def ld_group(g):
        base = g * GROUP
        return tuple(
            r[pl.ds(base + k * L, L)]
            for r in (x_v, y_v, z_v)
            for k in range(UNROLL))

    def outer(gi, t, carry):
        cxs, cys, czs, farg, vec = carry
        vec = jnp.where(lanes == t, jnp.full((L,), farg, jnp.int32), vec)

        def inner(j, icarry):
            m, mi, cur = icarry
            nxt = ld_group(j + 1)
            base = j * GROUP
            for k in range(UNROLL):
                sl = pl.ds(base + k * L, L)
                dx = cur[k] - cxs
                dy = cur[UNROLL + k] - cys
                dz = cur[2 * UNROLL + k] - czs
                d = dx * dx + dy * dy + dz * dz
                t2 = jnp.minimum(t_v[sl], d)
                t_v[sl] = t2
                gt = t2 > m
                m = jnp.where(gt, t2, m)
                mi = jnp.where(gt, base + k * L + lanes, mi)
            return m, mi, nxt

        m0 = jnp.full((L,), NEG_HUGE, jnp.float32)
        mi0 = jnp.zeros((L,), jnp.int32)
        m, mi, _ = lax.fori_loop(
            0, STEPS, inner, (m0, mi0, ld_group(0)))

        mx = jnp.max(m)
        lmin = jnp.min(jnp.where(m == mx, mi, BIG_I32))
        gidx = lmin + off
        win = mi == lmin
        nh = jnp.full((L,), NEG_HUGE, jnp.float32)
        px = jnp.max(jnp.where(win, plsc.load_gather(x_v, [mi]), nh))
        py = jnp.max(jnp.where(win, plsc.load_gather(y_v, [mi]), nh))
        pz = jnp.max(jnp.where(win, plsc.load_gather(z_v, [mi]), nh))

        par = gi % 2
        gf = plsc.bitcast(jnp.full((L,), gidx, jnp.int32), jnp.float32)
        msg = jnp.where(lanes == 0, jnp.full((L,), mx, jnp.float32),
              jnp.where(lanes == 1, gf,
              jnp.where(lanes == 2, jnp.full((L,), px, jnp.float32),
              jnp.where(lanes == 3, jnp.full((L,), py, jnp.float32),
                        jnp.full((L,), pz, jnp.float32)))))
        msg_v[...] = msg
        pltpu.sync_copy(msg_v, shr.at[pl.ds(par * (16 * L) + s * L, L)])
        plsc.fetch_and_add(cnt.at[0], jnp.int32(1), subcore_id=partner)

        lax.while_loop(
            lambda v: v <= gi,
            lambda v: plsc.fetch_and_add(cnt.at[0], jnp.int32(0),
                                         subcore_id=s),
            jnp.int32(0))

        pltpu.sync_copy(shr.at[pl.ds(par * (16 * L) + partner * L, L)],
                        rcv_v)
        r = rcv_v[...]
        ri = plsc.bitcast(r, jnp.int32)

        m2 = r[0]
        g2 = ri[1]
        take2 = (m2 > mx) | ((m2 == mx) & (g2 < gidx))
        farg2 = jnp.where(take2, g2, gidx)
        cx2 = jnp.full((L,), jnp.where(take2, r[2], px), jnp.float32)
        cy2 = jnp.full((L,), jnp.where(take2, r[3], py), jnp.float32)
        cz2 = jnp.full((L,), jnp.where(take2, r[4], pz), jnp.float32)
        return cx2, cy2, cz2, farg2, vec

    def blk_body(blk, carry):
        cxs, cys, czs, farg = carry

        def sub(t, sc):
            return outer(blk * L + t, t, sc)

        cxs, cys, czs, farg, vec = lax.fori_loop(
            0, L, sub,
            (cxs, cys, czs, farg, jnp.zeros((L,), jnp.int32)))

        @pl.when(half == 0)
        def _():
            i_v[pl.ds(blk * L, L)] = vec

        return cxs, cys, czs, farg

    carry0 = (jnp.full((L,), cx0, jnp.float32),
              jnp.full((L,), cy0, jnp.float32),
              jnp.full((L,), cz0, jnp.float32),
              jnp.int32(0))
    lax.fori_loop(0, NPOINT // L, blk_body, carry0)

    @pl.when(half == 0)
    def _():
        pltpu.sync_copy(i_v, idx_hbm.at[b])


_fps = functools.partial(
    pl.kernel,
    out_type=jax.ShapeDtypeStruct((B, NPOINT), jnp.int32),
    mesh=plsc.VectorSubcoreMesh(core_axis_name="c", subcore_axis_name="s"),
    compiler_params=pltpu.CompilerParams(needs_layout_passes=False),
    scratch_types=[
        pltpu.VMEM((H + GROUP,), jnp.float32),
        pltpu.VMEM((H + GROUP,), jnp.float32),
        pltpu.VMEM((H + GROUP,), jnp.float32),
        pltpu.VMEM((H + GROUP,), jnp.float32),
        pltpu.VMEM((NPOINT,), jnp.int32),
        pltpu.VMEM((L,), jnp.float32),
        pltpu.VMEM((L,), jnp.float32),
        pltpu.VMEM_SHARED((2 * 16 * L,), jnp.float32),
        pltpu.SMEM((1,), jnp.int32),
    ],
)(_fps_body)


def kernel(xyz, npoint):
    del npoint
    x = xyz[:, :, 0]
    y = xyz[:, :, 1]
    z = xyz[:, :, 2]
    return _fps(x, y, z)


# speedup vs baseline: 1.4309x; 1.0166x over previous
import functools

import jax
import jax.numpy as jnp
from jax import lax
from jax.experimental import pallas as pl
from jax.experimental.pallas import tpu as pltpu
from jax.experimental.pallas import tpu_sc as plsc

B = 16
N = 16384
H = N // 2
NPOINT = 512
L = 16
UNROLL = 8
GROUP = L * UNROLL
STEPS = H // GROUP
BIG_I32 = 2**31 - 1
NEG_HUGE = -3.4e38


def _fps_body(x_hbm, y_hbm, z_hbm, idx_hbm, x_v, y_v, z_v, t_v, i_v,
              msg_v, rcv_v, shr, cnt):
    c = lax.axis_index("c")
    s = lax.axis_index("s")
    b = c * 8 + s // 2
    half = s % 2
    partner = s ^ 1
    off = half * H

    cnt[0] = jnp.int32(0)

    pltpu.sync_copy(x_hbm.at[b, pl.ds(off, H)], x_v.at[pl.ds(0, H)])
    pltpu.sync_copy(y_hbm.at[b, pl.ds(off, H)], y_v.at[pl.ds(0, H)])
    pltpu.sync_copy(z_hbm.at[b, pl.ds(off, H)], z_v.at[pl.ds(0, H)])

    lanes = lax.iota(jnp.int32, L)
    neg = jnp.full((L,), -1e10, jnp.float32)
    pos = jnp.full((L,), 1e10, jnp.float32)
    zero = jnp.zeros((L,), jnp.float32)

    for k in range(UNROLL):
        sl = pl.ds(H + k * L, L)
        x_v[sl] = zero
        y_v[sl] = zero
        z_v[sl] = zero
        t_v[sl] = zero

    def init_body(j, carry):
        sl = pl.ds(j * L, L)
        xs = x_v[sl]
        ys = y_v[sl]
        zs = z_v[sl]
        mag = xs * xs + ys * ys + zs * zs
        t_v[sl] = jnp.where(mag <= 0.001, neg, pos)
        return carry

    lax.fori_loop(0, H // L, init_body, 0)

    pltpu.sync_copy(x_hbm.at[b, pl.ds(0, L)], msg_v)
    cx0 = msg_v[...][0]
    pltpu.sync_copy(y_hbm.at[b, pl.ds(0, L)], msg_v)
    cy0 = msg_v[...][0]
    pltpu.sync_copy(z_hbm.at[b, pl.ds(0, L)], msg_v)
    cz0 = msg_v[...][0]

    def ld_group(g):
        base = g * GROUP
        return tuple(
            r[pl.ds(base + k * L, L)]
            for r in (x_v, y_v, z_v)
            for k in range(UNROLL))

    def outer(gi, t, carry):
        cxs, cys, czs, farg, vec = carry
        vec = jnp.where(lanes == t, jnp.full((L,), farg, jnp.int32), vec)

        def inner(j, icarry):
            m, mi, cur = icarry
            nxt = ld_group(j + 1)
            base = j * GROUP
            for k in range(UNROLL):
                sl = pl.ds(base + k * L, L)
                dx = cur[k] - cxs
                dy = cur[UNROLL + k] - cys
                dz = cur[2 * UNROLL + k] - czs
                d = dx * dx + dy * dy + dz * dz
                t2 = jnp.minimum(t_v[sl], d)
                t_v[sl] = t2
                gt = t2 > m
                m = jnp.where(gt, t2, m)
                mi = jnp.where(gt, base + k * L + lanes, mi)
            return m, mi, nxt

        m0 = jnp.full((L,), NEG_HUGE, jnp.float32)
        mi0 = jnp.zeros((L,), jnp.int32)
        m, mi, _ = lax.fori_loop(
            0, STEPS, inner, (m0, mi0, ld_group(0)))

        mx = jnp.max(m)
        lmin = jnp.min(jnp.where(m == mx, mi, BIG_I32))
        gidx = lmin + off
        win = mi == lmin
        nh = jnp.full((L,), NEG_HUGE, jnp.float32)
        px = jnp.max(jnp.where(win, plsc.load_gather(x_v, [mi]), nh))
        py = jnp.max(jnp.where(win, plsc.load_gather(y_v, [mi]), nh))
        pz = jnp.max(jnp.where(win, plsc.load_gather(z_v, [mi]), nh))

        par = gi % 2
        gf = plsc.bitcast(jnp.full((L,), gidx, jnp.int32), jnp.float32)
        msg = jnp.where(lanes == 0, jnp.full((L,), mx, jnp.float32),
              jnp.where(lanes == 1, gf,
              jnp.where(lanes == 2, jnp.full((L,), px, jnp.float32),
              jnp.where(lanes == 3, jnp.full((L,), py, jnp.float32),
                        jnp.full((L,), pz, jnp.float32)))))
        msg_v[...] = msg
        pltpu.sync_copy(msg_v, shr.at[pl.ds(par * (16 * L) + s * L, L)])
        plsc.fetch_and_add(cnt.at[0], jnp.int32(1), subcore_id=partner)

        lax.while_loop(
            lambda v: v <= gi,
            lambda v: plsc.fetch_and_add(cnt.at[0], jnp.int32(0),
                                         subcore_id=s),
            jnp.int32(0))

        pltpu.sync_copy(shr.at[pl.ds(par * (16 * L) + partner * L, L)],
                        rcv_v)
        r = rcv_v[...]
        ri = plsc.bitcast(r, jnp.int32)

        m2 = r[0]
        g2 = ri[1]
        take2 = (m2 > mx) | ((m2 == mx) & (g2 < gidx))
        farg2 = jnp.where(take2, g2, gidx)
        cx2 = jnp.full((L,), jnp.where(take2, r[2], px), jnp.float32)
        cy2 = jnp.full((L,), jnp.where(take2, r[3], py), jnp.float32)
        cz2 = jnp.full((L,), jnp.where(take2, r[4], pz), jnp.float32)
        return cx2, cy2, cz2, farg2, vec

    def blk_body(blk, carry):
        cxs, cys, czs, farg = carry

        def sub(t, sc):
            return outer(blk * L + t, t, sc)

        cxs, cys, czs, farg, vec = lax.fori_loop(
            0, L, sub,
            (cxs, cys, czs, farg, jnp.zeros((L,), jnp.int32)))

        @pl.when(half == 0)
        def _():
            i_v[pl.ds(blk * L, L)] = vec

        return cxs, cys, czs, farg

    carry0 = (jnp.full((L,), cx0, jnp.float32),
              jnp.full((L,), cy0, jnp.float32),
              jnp.full((L,), cz0, jnp.float32),
              jnp.int32(0))
    lax.fori_loop(0, NPOINT // L, blk_body, carry0)

    @pl.when(half == 0)
    def _():
        pltpu.sync_copy(i_v, idx_hbm.at[b])


_fps = functools.partial(
    pl.kernel,
    out_type=jax.ShapeDtypeStruct((B, NPOINT), jnp.int32),
    mesh=plsc.VectorSubcoreMesh(core_axis_name="c", subcore_axis_name="s"),
    compiler_params=pltpu.CompilerParams(needs_layout_passes=False),
    scratch_types=[
        pltpu.VMEM((H + GROUP,), jnp.float32),
        pltpu.VMEM((H + GROUP,), jnp.float32),
        pltpu.VMEM((H + GROUP,), jnp.float32),
        pltpu.VMEM((H + GROUP,), jnp.float32),
        pltpu.VMEM((NPOINT,), jnp.int32),
        pltpu.VMEM((L,), jnp.float32),
        pltpu.VMEM((L,), jnp.float32),
        pltpu.VMEM_SHARED((2 * 16 * L,), jnp.float32),
        pltpu.SMEM((1,), jnp.int32),
    ],
)(_fps_body)


def kernel(xyz, npoint):
    del npoint
    x = xyz[:, :, 0]
    y = xyz[:, :, 1]
    z = xyz[:, :, 2]
    return _fps(x, y, z)
